# baseline (speedup 1.0000x reference)
import functools

import jax
import jax.numpy as jnp
from jax import lax
from jax.experimental import pallas as pl
from jax.experimental.pallas import tpu as pltpu
from jax.experimental.pallas import tpu_sc as plsc

B = 4096
L = 200
D = 64
NR = 4
NBUF = 3
NOB = 3

_info = plsc.get_sparse_core_info()
NC = _info.num_cores
NS = _info.num_subcores
NW = NC * NS
BW = B // NW
ROWS_PER_TILE = (NR * L) // NS

_mesh = plsc.VectorSubcoreMesh(core_axis_name="c", subcore_axis_name="s")


@functools.partial(
    pl.kernel,
    out_type=(
        jax.ShapeDtypeStruct((L, D // 8, B // BW, 8, BW), jnp.float32),
        jax.ShapeDtypeStruct((L, D // 8, B // BW, 8, BW), jnp.float32),
        jax.ShapeDtypeStruct((NC * NR * L, D), jnp.float32),
    ),
    mesh=_mesh,
    scratch_types=[
        pltpu.VMEM((NBUF * BW,), jnp.int32),
        pltpu.VMEM((NBUF * BW,), jnp.int32),
        pltpu.VMEM((NBUF * BW,), jnp.int32),
        pltpu.VMEM((NBUF, BW, D), jnp.float32),
        pltpu.VMEM((NBUF, BW, D), jnp.float32),
        pltpu.VMEM((NOB, D // 8, 1, 8, BW), jnp.float32),
        pltpu.VMEM((NOB, D // 8, 1, 8, BW), jnp.float32),
        pltpu.VMEM((L, D), jnp.float32),
        pltpu.VMEM((NR, D), jnp.float32),
        pltpu.VMEM((ROWS_PER_TILE, D), jnp.float32),
    ]
    + [pltpu.SemaphoreType.DMA] * (3 * NBUF + 1),
    compiler_params=pltpu.CompilerParams(use_tc_tiling_on_sc=False,
                                         needs_layout_passes=False),
)
def _emb_kernel(eidx_hbm, ridx_hbm, etab_hbm, rtab_hbm, pos_hbm,
                enc_hbm, dec_hbm, comb_hbm,
                eidx_v, ridx_v, didx_v, gbuf_e, gbuf_d, obuf_e, obuf_d,
                pos_v, resp_v, build_v, *sems):
    sem_i = sems[0:NBUF]
    sem_g = sems[NBUF:2 * NBUF]
    sem_w = sems[2 * NBUF:3 * NBUF]
    sem_misc = sems[3 * NBUF]

    cid = lax.axis_index("c")
    sid = lax.axis_index("s")
    wid = sid * NC + cid
    b0 = wid * BW

    pltpu.async_copy(pos_hbm, pos_v, sem_misc)
    pltpu.async_copy(rtab_hbm, resp_v, sem_misc)
    pltpu.make_async_copy(pos_hbm, pos_v, sem_misc).wait()
    pltpu.make_async_copy(rtab_hbm, resp_v, sem_misc).wait()

    comb_base = cid * (NR * L)

    r_own = sid // (L // ROWS_PER_TILE)
    l_own = (sid % (L // ROWS_PER_TILE)) * ROWS_PER_TILE
    rvec = [resp_v[r_own, pl.ds(c * 16, 16)] for c in range(D // 16)]

    @pl.loop(0, ROWS_PER_TILE)
    def _build(l):
        for c in range(D // 16):
            sl = pl.ds(c * 16, 16)
            build_v[l, sl] = pos_v[l_own + l, sl] + rvec[c]

    pltpu.sync_copy(
        build_v, comb_hbm.at[pl.ds(comb_base + sid * ROWS_PER_TILE, ROWS_PER_TILE)])
    plsc.subcore_barrier()

    rj = [lax.iota(jnp.int32, 16) + 16 * j for j in range(BW // 16)]

    def issue_idx(l, s):
        pltpu.async_copy(eidx_hbm.at[l, pl.ds(b0, BW)],
                         eidx_v.at[pl.ds(s * BW, BW)], sem_i[s])
        pltpu.async_copy(ridx_hbm.at[l, pl.ds(b0, BW)],
                         ridx_v.at[pl.ds(s * BW, BW)], sem_i[s])

    def drain_idx(l, s):
        pltpu.make_async_copy(eidx_hbm.at[l, pl.ds(b0, BW)],
                              eidx_v.at[pl.ds(s * BW, BW)], sem_i[s]).wait()
        pltpu.make_async_copy(ridx_hbm.at[l, pl.ds(b0, BW)],
                              ridx_v.at[pl.ds(s * BW, BW)], sem_i[s]).wait()

    def compute_didx(l, s):
        base = l + comb_base
        for i in range(BW // 16):
            sl = pl.ds(s * BW + 16 * i, 16)
            didx_v[sl] = ridx_v[sl] * L + base

    def issue_gather(s):
        pltpu.async_copy(etab_hbm.at[eidx_v.at[pl.ds(s * BW, BW)]],
                         gbuf_e.at[s], sem_g[s])
        pltpu.async_copy(comb_hbm.at[didx_v.at[pl.ds(s * BW, BW)]],
                         gbuf_d.at[s], sem_g[s])

    def drain_gather(s):
        pltpu.make_async_copy(etab_hbm.at[eidx_v.at[pl.ds(s * BW, BW)]],
                              gbuf_e.at[s], sem_g[s]).wait()
        pltpu.make_async_copy(comb_hbm.at[didx_v.at[pl.ds(s * BW, BW)]],
                              gbuf_d.at[s], sem_g[s]).wait()

    def transpose_chunk(l, s, bo):
        lsplat = lax.full_like(rj[0], l)

        @pl.loop(0, D)
        def _t(d):
            dhi = lax.shift_right_logical(d, 3)
            dlo = lax.bitwise_and(d, 7)
            dsplat = lax.full_like(rj[0], d)
            pe = plsc.load_gather(pos_v, [lsplat, dsplat])
            for j in range(BW // 16):
                sl = pl.ds(16 * j, 16)
                ve = plsc.load_gather(gbuf_e.at[s], [rj[j], dsplat])
                obuf_e[bo, dhi, 0, dlo, sl] = ve + pe
                vd = plsc.load_gather(gbuf_d.at[s], [rj[j], dsplat])
                obuf_d[bo, dhi, 0, dlo, sl] = vd

    def issue_wb(l, bo):
        pltpu.async_copy(obuf_e.at[bo],
                         enc_hbm.at[l, pl.ds(0, D // 8), pl.ds(wid, 1)], sem_w[bo])
        pltpu.async_copy(obuf_d.at[bo],
                         dec_hbm.at[l, pl.ds(0, D // 8), pl.ds(wid, 1)], sem_w[bo])

    def drain_wb(l, bo):
        pltpu.make_async_copy(obuf_e.at[bo],
                              enc_hbm.at[l, pl.ds(0, D // 8), pl.ds(wid, 1)],
                              sem_w[bo]).wait()
        pltpu.make_async_copy(obuf_d.at[bo],
                              dec_hbm.at[l, pl.ds(0, D // 8), pl.ds(wid, 1)],
                              sem_w[bo]).wait()

    for l in range(NBUF):
        issue_idx(l, l)
    for l in range(2):
        drain_idx(l, l)
        compute_didx(l, l)
        issue_gather(l)

    @pl.loop(0, L)
    def _main(l):
        for s in range(NBUF):
            s2 = (s + 2) % NBUF

            @pl.when(lax.rem(l, NBUF) == s)
            def _body():
                @pl.when(l + 2 < L)
                def _():
                    drain_idx(l + 2, s2)
                    compute_didx(l + 2, s2)
                    issue_gather(s2)

                drain_gather(s)

                @pl.when(l + NBUF < L)
                def _():
                    issue_idx(l + NBUF, s)

                @pl.when(l >= NOB)
                def _():
                    drain_wb(l - NOB, s)

                transpose_chunk(l, s, s)
                issue_wb(l, s)

    for l in range(L - NOB, L):
        drain_wb(l, l % NOB)


def kernel(input_exercise, input_r, exercise_table, response_table, position_table):
    enc5, dec5, _ = _emb_kernel(input_exercise.T, input_r.T, exercise_table,
                                response_table, position_table)
    enc = jnp.transpose(enc5, (2, 4, 0, 1, 3)).reshape(B, L, D)
    dec = jnp.transpose(dec5, (2, 4, 0, 1, 3)).reshape(B, L, D)
    return enc, dec


# speedup vs baseline: 1.2839x; 1.2839x over previous
import functools

import jax
import jax.numpy as jnp
from jax import lax
from jax.experimental import pallas as pl
from jax.experimental.pallas import tpu as pltpu
from jax.experimental.pallas import tpu_sc as plsc

B = 4096
L = 200
D = 64
NR = 4
NBUF = 3
NOB = 3

_info = plsc.get_sparse_core_info()
NC = _info.num_cores
NS = _info.num_subcores
NW = NC * NS
BW = B // NW
ROWS_PER_TILE = (NR * L) // NS

_mesh = plsc.VectorSubcoreMesh(core_axis_name="c", subcore_axis_name="s")


@functools.partial(
    pl.kernel,
    out_type=(
        jax.ShapeDtypeStruct((L, D // 8, B // BW, 8, BW), jnp.float32),
        jax.ShapeDtypeStruct((L, D // 8, B // BW, 8, BW), jnp.float32),
        jax.ShapeDtypeStruct((NC * NR * L, D), jnp.float32),
    ),
    mesh=_mesh,
    scratch_types=[
        pltpu.VMEM((NBUF * BW,), jnp.int32),
        pltpu.VMEM((NBUF * BW,), jnp.int32),
        pltpu.VMEM((NBUF * BW,), jnp.int32),
        pltpu.VMEM((NBUF, BW, D), jnp.float32),
        pltpu.VMEM((NBUF, BW, D), jnp.float32),
        pltpu.VMEM((BW, D + 8), jnp.float32),
        pltpu.VMEM((NOB, D // 8, 1, 8, BW), jnp.float32),
        pltpu.VMEM((NOB, D // 8, 1, 8, BW), jnp.float32),
        pltpu.VMEM((L, D), jnp.float32),
        pltpu.VMEM((NR, D), jnp.float32),
        pltpu.VMEM((ROWS_PER_TILE, D), jnp.float32),
    ]
    + [pltpu.SemaphoreType.DMA] * (3 * NBUF + 1),
    compiler_params=pltpu.CompilerParams(use_tc_tiling_on_sc=False,
                                         needs_layout_passes=False),
)
def _emb_kernel(eidx_hbm, ridx_hbm, etab_hbm, rtab_hbm, pos_hbm,
                enc_hbm, dec_hbm, comb_hbm,
                eidx_v, ridx_v, didx_v, gbuf_e, gbuf_d, tbuf, obuf_e, obuf_d,
                pos_v, resp_v, build_v, *sems):
    sem_i = sems[0:NBUF]
    sem_g = sems[NBUF:2 * NBUF]
    sem_w = sems[2 * NBUF:3 * NBUF]
    sem_misc = sems[3 * NBUF]

    cid = lax.axis_index("c")
    sid = lax.axis_index("s")
    wid = sid * NC + cid
    b0 = wid * BW

    pltpu.async_copy(pos_hbm, pos_v, sem_misc)
    pltpu.async_copy(rtab_hbm, resp_v, sem_misc)
    pltpu.make_async_copy(pos_hbm, pos_v, sem_misc).wait()
    pltpu.make_async_copy(rtab_hbm, resp_v, sem_misc).wait()

    comb_base = cid * (NR * L)

    r_own = sid // (L // ROWS_PER_TILE)
    l_own = (sid % (L // ROWS_PER_TILE)) * ROWS_PER_TILE
    rvec = [resp_v[r_own, pl.ds(c * 16, 16)] for c in range(D // 16)]

    @pl.loop(0, ROWS_PER_TILE)
    def _build(l):
        for c in range(D // 16):
            sl = pl.ds(c * 16, 16)
            build_v[l, sl] = pos_v[l_own + l, sl] + rvec[c]

    pltpu.sync_copy(
        build_v, comb_hbm.at[pl.ds(comb_base + sid * ROWS_PER_TILE, ROWS_PER_TILE)])
    plsc.subcore_barrier()

    rj = [lax.iota(jnp.int32, 16) + 16 * j for j in range(BW // 16)]

    def issue_idx(l, s):
        pltpu.async_copy(eidx_hbm.at[l, pl.ds(b0, BW)],
                         eidx_v.at[pl.ds(s * BW, BW)], sem_i[s])
        pltpu.async_copy(ridx_hbm.at[l, pl.ds(b0, BW)],
                         ridx_v.at[pl.ds(s * BW, BW)], sem_i[s])

    def drain_idx(l, s):
        pltpu.make_async_copy(eidx_hbm.at[l, pl.ds(b0, BW)],
                              eidx_v.at[pl.ds(s * BW, BW)], sem_i[s]).wait()
        pltpu.make_async_copy(ridx_hbm.at[l, pl.ds(b0, BW)],
                              ridx_v.at[pl.ds(s * BW, BW)], sem_i[s]).wait()

    def compute_didx(l, s):
        base = l + comb_base
        for i in range(BW // 16):
            sl = pl.ds(s * BW + 16 * i, 16)
            didx_v[sl] = ridx_v[sl] * L + base

    def issue_gather(s):
        pltpu.async_copy(etab_hbm.at[eidx_v.at[pl.ds(s * BW, BW)]],
                         gbuf_e.at[s], sem_g[s])
        pltpu.async_copy(comb_hbm.at[didx_v.at[pl.ds(s * BW, BW)]],
                         gbuf_d.at[s], sem_g[s])

    def drain_gather(s):
        pltpu.make_async_copy(etab_hbm.at[eidx_v.at[pl.ds(s * BW, BW)]],
                              gbuf_e.at[s], sem_g[s]).wait()
        pltpu.make_async_copy(comb_hbm.at[didx_v.at[pl.ds(s * BW, BW)]],
                              gbuf_d.at[s], sem_g[s]).wait()

    def transpose_chunk(l, s, bo):
        prow = [pos_v[l, pl.ds(16 * c, 16)] for c in range(D // 16)]

        @pl.loop(0, BW, unroll=4)
        def _cpe(b):
            for c in range(D // 16):
                sl = pl.ds(16 * c, 16)
                tbuf[b, sl] = gbuf_e[s, b, sl] + prow[c]

        @pl.loop(0, D // 8)
        def _te(dhi):
            for dlo in range(8):
                dsplat = lax.full_like(rj[0], dhi * 8 + dlo)
                for j in range(BW // 16):
                    v = plsc.load_gather(tbuf, [rj[j], dsplat])
                    obuf_e[bo, dhi, 0, dlo, pl.ds(16 * j, 16)] = v

        @pl.loop(0, BW, unroll=4)
        def _cpd(b):
            for c in range(D // 16):
                sl = pl.ds(16 * c, 16)
                tbuf[b, sl] = gbuf_d[s, b, sl]

        @pl.loop(0, D // 8)
        def _td(dhi):
            for dlo in range(8):
                dsplat = lax.full_like(rj[0], dhi * 8 + dlo)
                for j in range(BW // 16):
                    v = plsc.load_gather(tbuf, [rj[j], dsplat])
                    obuf_d[bo, dhi, 0, dlo, pl.ds(16 * j, 16)] = v

    def issue_wb(l, bo):
        pltpu.async_copy(obuf_e.at[bo],
                         enc_hbm.at[l, pl.ds(0, D // 8), pl.ds(wid, 1)], sem_w[bo])
        pltpu.async_copy(obuf_d.at[bo],
                         dec_hbm.at[l, pl.ds(0, D // 8), pl.ds(wid, 1)], sem_w[bo])

    def drain_wb(l, bo):
        pltpu.make_async_copy(obuf_e.at[bo],
                              enc_hbm.at[l, pl.ds(0, D // 8), pl.ds(wid, 1)],
                              sem_w[bo]).wait()
        pltpu.make_async_copy(obuf_d.at[bo],
                              dec_hbm.at[l, pl.ds(0, D // 8), pl.ds(wid, 1)],
                              sem_w[bo]).wait()

    for l in range(NBUF):
        issue_idx(l, l)
    for l in range(2):
        drain_idx(l, l)
        compute_didx(l, l)
        issue_gather(l)

    @pl.loop(0, L)
    def _main(l):
        for s in range(NBUF):
            s2 = (s + 2) % NBUF

            @pl.when(lax.rem(l, NBUF) == s)
            def _body():
                @pl.when(l + 2 < L)
                def _():
                    drain_idx(l + 2, s2)
                    compute_didx(l + 2, s2)
                    issue_gather(s2)

                drain_gather(s)

                @pl.when(l + NBUF < L)
                def _():
                    issue_idx(l + NBUF, s)

                @pl.when(l >= NOB)
                def _():
                    drain_wb(l - NOB, s)

                transpose_chunk(l, s, s)
                issue_wb(l, s)

    for l in range(L - NOB, L):
        drain_wb(l, l % NOB)


def kernel(input_exercise, input_r, exercise_table, response_table, position_table):
    enc5, dec5, _ = _emb_kernel(input_exercise.T, input_r.T, exercise_table,
                                response_table, position_table)
    enc = jnp.transpose(enc5, (2, 4, 0, 1, 3)).reshape(B, L, D)
    dec = jnp.transpose(dec5, (2, 4, 0, 1, 3)).reshape(B, L, D)
    return enc, dec


# speedup vs baseline: 2.0239x; 1.5763x over previous
import functools

import jax
import jax.numpy as jnp
from jax import lax
from jax.experimental import pallas as pl
from jax.experimental.pallas import tpu as pltpu
from jax.experimental.pallas import tpu_sc as plsc

B = 4096
L = 200
D = 64
NR = 4
LP = 256
NBUF = 4
NVI = LP // 16
SLICES = ((0, 80), (80, 80), (160, 40))

_info = plsc.get_sparse_core_info()
NC = _info.num_cores
NS = _info.num_subcores
NW = NC * NS
ROWS_PER_W = B // NW
ROWS_PER_TILE = (NR * L) // NS

_mesh = plsc.VectorSubcoreMesh(core_axis_name="c", subcore_axis_name="s")


@functools.partial(
    pl.kernel,
    out_type=(
        jax.ShapeDtypeStruct((B, L, D), jnp.float32),
        jax.ShapeDtypeStruct((B, L, D), jnp.float32),
        jax.ShapeDtypeStruct((NC * NR * L, D), jnp.float32),
    ),
    mesh=_mesh,
    scratch_types=[
        pltpu.VMEM((NBUF * LP,), jnp.int32),
        pltpu.VMEM((NBUF * LP,), jnp.int32),
        pltpu.VMEM((NBUF * LP,), jnp.int32),
        pltpu.VMEM((NBUF, L, D), jnp.float32),
        pltpu.VMEM((NBUF, L, D), jnp.float32),
        pltpu.VMEM((L, D), jnp.float32),
        pltpu.VMEM((LP,), jnp.int32),
        pltpu.VMEM((NR, D), jnp.float32),
        pltpu.VMEM((ROWS_PER_TILE, D), jnp.float32),
    ]
    + [pltpu.SemaphoreType.DMA] * (3 * NBUF + 1),
    compiler_params=pltpu.CompilerParams(use_tc_tiling_on_sc=False),
)
def _emb_kernel(eidx_hbm, ridx_hbm, etab_hbm, rtab_hbm, pos_hbm,
                enc_hbm, dec_hbm, comb_hbm,
                eidx_v, ridx_v, didx_v, enc_v, dec_v,
                pos_v, lpos_v, resp_v, build_v, *sems):
    sem_i = sems[0:NBUF]
    sem_g = sems[NBUF:2 * NBUF]
    sem_w = sems[2 * NBUF:3 * NBUF]
    sem_misc = sems[3 * NBUF]

    cid = lax.axis_index("c")
    sid = lax.axis_index("s")
    wid = sid * NC + cid
    row0 = wid * ROWS_PER_W

    pltpu.async_copy(pos_hbm, pos_v, sem_misc)
    pltpu.async_copy(rtab_hbm, resp_v, sem_misc)
    pltpu.make_async_copy(pos_hbm, pos_v, sem_misc).wait()
    pltpu.make_async_copy(rtab_hbm, resp_v, sem_misc).wait()

    comb_base = cid * (NR * L)
    for i in range(NVI):
        lpos_v[pl.ds(16 * i, 16)] = lax.rem(
            jnp.full((16,), 16 * i, jnp.int32) + lax.iota(jnp.int32, 16), L
        ) + comb_base


    r_own = sid // (L // ROWS_PER_TILE)
    l_own = (sid % (L // ROWS_PER_TILE)) * ROWS_PER_TILE
    rvec = [resp_v[r_own, pl.ds(c * 16, 16)] for c in range(D // 16)]

    @pl.loop(0, ROWS_PER_TILE)
    def _build(l):
        for c in range(D // 16):
            sl = pl.ds(c * 16, 16)
            build_v[l, sl] = pos_v[l_own + l, sl] + rvec[c]

    pltpu.sync_copy(
        build_v, comb_hbm.at[pl.ds(comb_base + sid * ROWS_PER_TILE, ROWS_PER_TILE)])
    plsc.subcore_barrier()

    def issue_idx(g, s):
        row = row0 + g
        pltpu.async_copy(eidx_hbm.at[row], eidx_v.at[pl.ds(s * LP, LP)], sem_i[s])
        pltpu.async_copy(ridx_hbm.at[row], ridx_v.at[pl.ds(s * LP, LP)], sem_i[s])

    def drain_idx(g, s):
        row = row0 + g
        pltpu.make_async_copy(eidx_hbm.at[row], eidx_v.at[pl.ds(s * LP, LP)], sem_i[s]).wait()
        pltpu.make_async_copy(ridx_hbm.at[row], ridx_v.at[pl.ds(s * LP, LP)], sem_i[s]).wait()

    def compute_didx(s):
        for i in range(NVI):
            sl = pl.ds(s * LP + 16 * i, 16)
            didx_v[sl] = ridx_v[sl] * L + lpos_v[pl.ds(16 * i, 16)]

    def issue_gather(s):
        for (o, n) in SLICES:
            pltpu.async_copy(etab_hbm.at[eidx_v.at[pl.ds(s * LP + o, n)]],
                             enc_v.at[s, pl.ds(o, n)], sem_g[s])
            pltpu.async_copy(comb_hbm.at[didx_v.at[pl.ds(s * LP + o, n)]],
                             dec_v.at[s, pl.ds(o, n)], sem_g[s])

    def drain_gather(s):
        for (o, n) in SLICES:
            pltpu.make_async_copy(etab_hbm.at[eidx_v.at[pl.ds(s * LP + o, n)]],
                                  enc_v.at[s, pl.ds(o, n)], sem_g[s]).wait()
            pltpu.make_async_copy(comb_hbm.at[didx_v.at[pl.ds(s * LP + o, n)]],
                                  dec_v.at[s, pl.ds(o, n)], sem_g[s]).wait()

    def issue_wb(g, s):
        row = row0 + g
        pltpu.async_copy(enc_v.at[s], enc_hbm.at[row], sem_w[s])
        pltpu.async_copy(dec_v.at[s], dec_hbm.at[row], sem_w[s])

    def drain_wb(g, s):
        row = row0 + g
        pltpu.make_async_copy(enc_v.at[s], enc_hbm.at[row], sem_w[s]).wait()
        pltpu.make_async_copy(dec_v.at[s], dec_hbm.at[row], sem_w[s]).wait()

    def compute_pos(s):
        @pl.loop(0, L)
        def _pos_add(l):
            for c in range(D // 16):
                sl = pl.ds(c * 16, 16)
                enc_v[s, l, sl] = enc_v[s, l, sl] + pos_v[l, sl]

    for g in range(NBUF):
        issue_idx(g, g)
    for g in range(2):
        drain_idx(g, g)
        compute_didx(g)
        issue_gather(g)

    @pl.loop(0, ROWS_PER_W)
    def _main(g):
        for b in range(NBUF):
            b2 = (b + 2) % NBUF

            @pl.when(lax.rem(g, NBUF) == b)
            def _body():
                @pl.when(g + 2 < ROWS_PER_W)
                def _():
                    drain_idx(g + 2, b2)
                    compute_didx(b2)

                    @pl.when(g >= 2)
                    def _():
                        drain_wb(g - 2, b2)

                    issue_gather(b2)

                drain_gather(b)

                @pl.when(g + NBUF < ROWS_PER_W)
                def _():
                    issue_idx(g + NBUF, b)

                compute_pos(b)
                issue_wb(g, b)

    for g in range(ROWS_PER_W - NBUF, ROWS_PER_W):
        drain_wb(g, g % NBUF)


def kernel(input_exercise, input_r, exercise_table, response_table, position_table):
    eidx = jnp.pad(input_exercise, ((0, 0), (0, LP - L)))
    ridx = jnp.pad(input_r, ((0, 0), (0, LP - L)))
    enc, dec, _ = _emb_kernel(eidx, ridx, exercise_table,
                              response_table, position_table)
    return enc, dec
